# baseline (speedup 1.0000x reference)
import jax
import jax.numpy as jnp
import numpy as np
from jax.experimental import pallas as pl
from jax.experimental.pallas import tpu as pltpu
from jax.experimental.shard_map import shard_map
from jax.sharding import Mesh, PartitionSpec as P

_INPUT_DIM = 768
_EXP_DIM = 8192
_POOL = 30
_SEL = 8
_KEEP = 0.05
_K = int(_EXP_DIM * _KEEP)
_CHUNK = _EXP_DIM // _SEL
_ROWS = 128

_IMIN = np.int32(-2**31)


def _select_kernel(x_ref, p_ref, o_ref, e_scr):
    j = pl.program_id(1)
    part = jax.lax.dot_general(
        x_ref[...], p_ref[...], (((1,), (1,)), ((), ())),
        preferred_element_type=jnp.float32)
    e_scr[:, pl.ds(j * _CHUNK, _CHUNK)] = part

    @pl.when(j == _SEL - 1)
    def _finish():
        e = e_scr[...]
        b = jax.lax.bitcast_convert_type(e, jnp.int32)
        key = b ^ (jax.lax.shift_right_arithmetic(b, 31) & np.int32(0x7FFFFFFF))

        R = e.shape[0]
        t_u = jnp.zeros((R, 1), jnp.int32)
        for i in range(32):
            sh = 31 - i
            bit = np.int32((1 << sh) if sh < 31 else int(_IMIN))
            cand_u = t_u | bit
            cand_s = cand_u ^ _IMIN
            cnt = jnp.sum((key >= cand_s).astype(jnp.int32), axis=1,
                          keepdims=True)
            t_u = jnp.where(cnt >= _K, cand_u, t_u)

        ts = t_u ^ _IMIN
        gt = key > ts
        eq = key == ts
        cnt_gt = jnp.sum(gt.astype(jnp.int32), axis=1, keepdims=True)
        kp = _K - cnt_gt

        idx = jax.lax.broadcasted_iota(jnp.int32, (R, _EXP_DIM), 1)
        eqi = eq.astype(jnp.int32)
        tau = jnp.zeros((R, 1), jnp.int32)
        for bb in range(12, -1, -1):
            cand = tau + (1 << bb)
            c = jnp.sum(jnp.where(idx < cand, eqi, 0), axis=1, keepdims=True)
            tau = jnp.where(c < kp, cand, tau)
        keep = gt | (eq & (idx <= tau))

        o_ref[...] = jnp.where(keep, e, 0.0)


def _sparse_code(x, projection):
    B = x.shape[0]
    return pl.pallas_call(
        _select_kernel,
        grid=(B // _ROWS, _SEL),
        in_specs=[
            pl.BlockSpec((_ROWS, _INPUT_DIM), lambda i, j: (i, 0)),
            pl.BlockSpec((_CHUNK, _INPUT_DIM), lambda i, j: (j, 0)),
        ],
        out_specs=pl.BlockSpec((_ROWS, _EXP_DIM), lambda i, j: (i, 0)),
        out_shape=jax.ShapeDtypeStruct((B, _EXP_DIM), jnp.float32),
        scratch_shapes=[pltpu.VMEM((_ROWS, _EXP_DIM), jnp.float32)],
    )(x, projection)


def _hash(sparse_code):
    indices = []
    for i in range(_SEL):
        start = i * _CHUNK
        chunk = sparse_code[:, start:start + _CHUNK]
        weights = jnp.arange(1, _CHUNK + 1, dtype=jnp.float32)
        weights = (weights * 2654435761.0) % _POOL
        hash_values = (chunk * weights[None, :]).sum(axis=1)
        prompt_idx = jnp.mod(hash_values, _POOL).astype(jnp.int32)
        indices.append(prompt_idx)
    return jnp.stack(indices, axis=1)


@jax.jit
def kernel(x, projection):
    B = x.shape[0]
    devs = jax.devices()
    if len(devs) >= 2 and B % (2 * _ROWS) == 0:
        mesh = Mesh(np.asarray(devs[:2]), ("d",))
        sparse_code = shard_map(
            _sparse_code, mesh=mesh,
            in_specs=(P("d", None), P(None, None)),
            out_specs=P("d", None), check_rep=False,
        )(x, projection)
        sparse_code = jax.lax.with_sharding_constraint(
            sparse_code, jax.sharding.NamedSharding(mesh, P()))
        return _hash(sparse_code)
    return _hash(_sparse_code(x, projection))


# speedup vs baseline: 1.1609x; 1.1609x over previous
import jax
import jax.numpy as jnp
import numpy as np
from jax.experimental import pallas as pl
from jax.experimental.pallas import tpu as pltpu
from jax.experimental.shard_map import shard_map
from jax.sharding import Mesh, PartitionSpec as P

_INPUT_DIM = 768
_EXP_DIM = 8192
_POOL = 30
_SEL = 8
_KEEP = 0.05
_K = int(_EXP_DIM * _KEEP)
_CHUNK = _EXP_DIM // _SEL
_ROWS = 128

_IMIN = np.int32(-2**31)


def _select_kernel(x_ref, p_ref, o_ref, e_scr):
    j = pl.program_id(1)
    part = jax.lax.dot_general(
        x_ref[...], p_ref[...], (((1,), (1,)), ((), ())),
        preferred_element_type=jnp.float32)
    e_scr[:, pl.ds(j * _CHUNK, _CHUNK)] = part

    @pl.when(j == _SEL - 1)
    def _finish():
        e = e_scr[...]
        b = jax.lax.bitcast_convert_type(e, jnp.int32)
        key = b ^ (jax.lax.shift_right_arithmetic(b, 31) & np.int32(0x7FFFFFFF))

        R = e.shape[0]
        t_u = jnp.zeros((R, 1), jnp.int32)
        for i in range(32):
            sh = 31 - i
            bit = np.int32((1 << sh) if sh < 31 else int(_IMIN))
            cand_u = t_u | bit
            cand_s = cand_u ^ _IMIN
            cnt = jnp.sum((key >= cand_s).astype(jnp.int32), axis=1,
                          keepdims=True)
            t_u = jnp.where(cnt >= _K, cand_u, t_u)

        ts = t_u ^ _IMIN
        gt = key > ts
        eq = key == ts
        cnt_gt = jnp.sum(gt.astype(jnp.int32), axis=1, keepdims=True)
        kp = _K - cnt_gt

        idx = jax.lax.broadcasted_iota(jnp.int32, (R, _EXP_DIM), 1)
        eqi = eq.astype(jnp.int32)
        tau = jnp.zeros((R, 1), jnp.int32)
        for bb in range(12, -1, -1):
            cand = tau + (1 << bb)
            c = jnp.sum(jnp.where(idx < cand, eqi, 0), axis=1, keepdims=True)
            tau = jnp.where(c < kp, cand, tau)
        keep = gt | (eq & (idx <= tau))

        o_ref[...] = jnp.where(keep, e, 0.0)


def _sparse_code(x, projection, out_rows=None):
    B = x.shape[0]
    out_rows = B if out_rows is None else out_rows
    return pl.pallas_call(
        _select_kernel,
        grid=(B // _ROWS, _SEL),
        in_specs=[
            pl.BlockSpec((_ROWS, _INPUT_DIM), lambda i, j: (i, 0)),
            pl.BlockSpec((_CHUNK, _INPUT_DIM), lambda i, j: (j, 0)),
        ],
        out_specs=pl.BlockSpec((_ROWS, _EXP_DIM), lambda i, j: (i, 0)),
        out_shape=jax.ShapeDtypeStruct((out_rows, _EXP_DIM), jnp.float32),
        scratch_shapes=[pltpu.VMEM((_ROWS, _EXP_DIM), jnp.float32)],
    )(x, projection)


def _hash(sparse_code):
    indices = []
    for i in range(_SEL):
        start = i * _CHUNK
        chunk = sparse_code[:, start:start + _CHUNK]
        weights = jnp.arange(1, _CHUNK + 1, dtype=jnp.float32)
        weights = (weights * 2654435761.0) % _POOL
        hash_values = (chunk * weights[None, :]).sum(axis=1)
        prompt_idx = jnp.mod(hash_values, _POOL).astype(jnp.int32)
        indices.append(prompt_idx)
    return jnp.stack(indices, axis=1)


@jax.jit
def kernel(x, projection):
    B = x.shape[0]
    devs = jax.devices()
    if len(devs) >= 2 and B % (2 * _ROWS) == 0:
        def _half(xs, proj):
            sc = _sparse_code(xs, proj, out_rows=B)
            return _hash(sc)[:xs.shape[0]]

        mesh = Mesh(np.asarray(devs[:2]), ("d",))
        return shard_map(
            _half, mesh=mesh,
            in_specs=(P("d", None), P(None, None)),
            out_specs=P("d", None), check_rep=False,
        )(x, projection)
    return _hash(_sparse_code(x, projection))
